# baseline (speedup 1.0000x reference)
import functools

import jax
import jax.numpy as jnp
from jax import lax
from jax.experimental import pallas as pl
from jax.experimental.pallas import tpu as pltpu
from jax.experimental.pallas import tpu_sc as plsc

N_GRID = 224
N_NODES = N_GRID * N_GRID
E_TOTAL = N_NODES * 16
HIDDEN = 96
LANES = 16
NC, NS = 2, 16
NW = NC * NS
E_PER_W = E_TOTAL // NW
GV = E_PER_W // LANES
ROWS = E_TOTAL // 128
ROWS_PER_W = ROWS // NW
N_PER_TILE = N_NODES // NS
NV = N_PER_TILE // LANES
SCH = 49
NCHUNKS = ROWS // NS // SCH
UPD_T = 14
UPD_CNT = N_NODES // UPD_T

_mesh = plsc.VectorSubcoreMesh(core_axis_name="c", subcore_axis_name="s")
_sc_params = pltpu.CompilerParams(needs_layout_passes=False)


@functools.partial(
    pl.kernel,
    out_type=[jax.ShapeDtypeStruct((E_TOTAL,), jnp.float32),
              jax.ShapeDtypeStruct((E_TOTAL,), jnp.float32)],
    mesh=_mesh,
    scratch_types=[pltpu.VMEM((N_NODES,), jnp.float32),
                   pltpu.VMEM((E_PER_W,), jnp.int32),
                   pltpu.VMEM((E_PER_W,), jnp.float32)],
    compiler_params=_sc_params,
)
def _sc_gather(h_hbm, esrc_hbm, edst_hbm, hs_out, hd_out, h_v, idx_v, out_v):
    wid = lax.axis_index("s") * NC + lax.axis_index("c")
    base = wid * E_PER_W
    pltpu.sync_copy(h_hbm, h_v)

    def one_pass(idx_hbm, o_hbm):
        pltpu.sync_copy(idx_hbm.at[pl.ds(base, E_PER_W)], idx_v)

        @plsc.parallel_loop(0, GV, unroll=8)
        def body(i):
            iv = idx_v[pl.ds(i * LANES, LANES)]
            out_v[pl.ds(i * LANES, LANES)] = plsc.load_gather(h_v, [iv])

        pltpu.sync_copy(out_v, o_hbm.at[pl.ds(base, E_PER_W)])

    one_pass(esrc_hbm, hs_out)
    one_pass(edst_hbm, hd_out)


@functools.partial(
    pl.kernel,
    out_type=[jax.ShapeDtypeStruct((E_TOTAL,), jnp.float32),
              jax.ShapeDtypeStruct((E_TOTAL,), jnp.float32),
              jax.ShapeDtypeStruct((NC, N_NODES), jnp.float32)],
    mesh=_mesh,
    scratch_types=[pltpu.VMEM((N_NODES,), jnp.float32),
                   pltpu.VMEM((E_PER_W,), jnp.int32),
                   pltpu.VMEM((E_PER_W,), jnp.float32),
                   pltpu.VMEM((SCH, 128), jnp.int32),
                   pltpu.VMEM((SCH, 128), jnp.float32),
                   pltpu.VMEM((2 * LANES,), jnp.float32),
                   pltpu.VMEM_SHARED((N_NODES,), jnp.float32)],
    compiler_params=_sc_params,
)
def _sc_mid(m_hbm, edst4_hbm, h_hbm, deg_hbm, esrc_hbm, edst_hbm, lslb_hbm,
            hs_out, hd_out, h1_out,
            h_v, idx_v, out_v, sidx_v, sval_v, par_v, acc_sh):
    cid = lax.axis_index("c")
    sid = lax.axis_index("s")
    nb = sid * N_PER_TILE

    @plsc.parallel_loop(0, NV, unroll=8)
    def zbody(i):
        out_v[pl.ds(i * LANES, LANES)] = jnp.zeros((LANES,), jnp.float32)

    pltpu.sync_copy(out_v.at[pl.ds(0, N_PER_TILE)], acc_sh.at[pl.ds(nb, N_PER_TILE)])
    plsc.subcore_barrier()

    for ci in range(NCHUNKS):
        blk = sid * NCHUNKS + ci
        pltpu.sync_copy(m_hbm.at[blk], sval_v)
        pltpu.sync_copy(edst4_hbm.at[blk], sidx_v)

        @plsc.parallel_loop(0, SCH, unroll=4)
        def sbody(j):
            pltpu.sync_copy(sval_v.at[j], acc_sh.at[sidx_v.at[j]], add=True)

    plsc.subcore_barrier()

    @pl.when(sid < UPD_T)
    def _update():
        ub = sid * UPD_CNT
        pltpu.sync_copy(acc_sh.at[pl.ds(ub, UPD_CNT)], out_v.at[pl.ds(0, UPD_CNT)])
        pltpu.sync_copy(h_hbm.at[pl.ds(ub, UPD_CNT)],
                        out_v.at[pl.ds(UPD_CNT, UPD_CNT)])
        pltpu.sync_copy(deg_hbm.at[pl.ds(ub, UPD_CNT)],
                        out_v.at[pl.ds(2 * UPD_CNT, UPD_CNT)])
        pltpu.sync_copy(lslb_hbm, par_v)
        ls = par_v[pl.ds(0, LANES)]
        lb = par_v[pl.ds(LANES, LANES)]

        @plsc.parallel_loop(0, UPD_CNT // LANES, unroll=4)
        def ubody(i):
            agg = out_v[pl.ds(i * LANES, LANES)]
            ho = out_v[pl.ds(UPD_CNT + i * LANES, LANES)]
            dg = out_v[pl.ds(2 * UPD_CNT + i * LANES, LANES)]
            x = ho + agg / dg
            mu = x
            dl = x - mu
            var = dl * dl
            v = var + 1e-6
            r = jnp.full((LANES,), 1000.0, jnp.float32)
            for _ in range(3):
                r = r * (1.5 - 0.5 * v * r * r)
            y = dl * r * ls + lb
            out_v[pl.ds(3 * UPD_CNT + i * LANES, LANES)] = y

        pltpu.sync_copy(out_v.at[pl.ds(3 * UPD_CNT, UPD_CNT)],
                        h1_out.at[cid].at[pl.ds(ub, UPD_CNT)])

    plsc.subcore_barrier()

    pltpu.sync_copy(h1_out.at[cid], h_v)
    wid = sid * NC + cid
    base = wid * E_PER_W

    def one_pass(i_hbm, o_hbm):
        pltpu.sync_copy(i_hbm.at[pl.ds(base, E_PER_W)], idx_v)

        @plsc.parallel_loop(0, GV, unroll=8)
        def body(i):
            iv = idx_v[pl.ds(i * LANES, LANES)]
            out_v[pl.ds(i * LANES, LANES)] = plsc.load_gather(h_v, [iv])

        pltpu.sync_copy(out_v, o_hbm.at[pl.ds(base, E_PER_W)])

    one_pass(esrc_hbm, hs_out)
    one_pass(edst_hbm, hd_out)


@functools.partial(
    pl.kernel,
    out_type=jax.ShapeDtypeStruct((NC, N_NODES), jnp.float32),
    mesh=_mesh,
    scratch_types=[pltpu.VMEM((ROWS_PER_W, 128), jnp.int32),
                   pltpu.VMEM((ROWS_PER_W, 128), jnp.float32),
                   pltpu.VMEM((N_PER_TILE,), jnp.float32),
                   pltpu.VMEM_SHARED((N_NODES,), jnp.float32)],
    compiler_params=_sc_params,
)
def _sc_scatter(m_hbm, edst_hbm, out_hbm, idx_v, val_v, zero_v, acc_sh):
    cid = lax.axis_index("c")
    sid = lax.axis_index("s")
    wid = sid * NC + cid

    @plsc.parallel_loop(0, NV, unroll=8)
    def zbody(i):
        zero_v[pl.ds(i * LANES, LANES)] = jnp.zeros((LANES,), jnp.float32)

    pltpu.sync_copy(zero_v, acc_sh.at[pl.ds(sid * N_PER_TILE, N_PER_TILE)])
    plsc.subcore_barrier()

    pltpu.sync_copy(m_hbm.at[wid], val_v)
    pltpu.sync_copy(edst_hbm.at[wid], idx_v)

    @plsc.parallel_loop(0, ROWS_PER_W, unroll=4)
    def sbody(j):
        pltpu.sync_copy(val_v.at[j], acc_sh.at[idx_v.at[j]], add=True)

    plsc.subcore_barrier()

    @pl.when(sid == 0)
    def _():
        pltpu.sync_copy(acc_sh, out_hbm.at[cid])


def _mlp_body(p_ref, hs_ref, hd_ref, w_ref, o_ref):
    def outer(s, c):
        sl = pl.ds(s * 16, 16)
        hs = hs_ref[sl, :]
        hd = hd_ref[sl, :]
        w = w_ref[sl, :]
        acc = jnp.zeros((16, 128), jnp.float32)
        for k in range(HIDDEN):
            t = (hs * p_ref[0, k] + hd * p_ref[1, k]
                 + w * p_ref[2, k] + p_ref[3, k])
            u = t + 0.044715 * (t * t * t)
            g = t * (1.0 + jnp.tanh(0.7978845608028654 * u))
            acc = acc + g * p_ref[4, k]
        o_ref[sl, :] = (acc + p_ref[5, 0]) * w
        return c

    lax.fori_loop(0, _MLP_BLK // 16, outer, 0)


_MLP_BLK = 448


def _tc_mlp(p, hs2, hd2, w2):
    return pl.pallas_call(
        _mlp_body,
        grid=(ROWS // _MLP_BLK,),
        in_specs=[
            pl.BlockSpec(memory_space=pltpu.SMEM),
            pl.BlockSpec((_MLP_BLK, 128), lambda i: (i, 0)),
            pl.BlockSpec((_MLP_BLK, 128), lambda i: (i, 0)),
            pl.BlockSpec((_MLP_BLK, 128), lambda i: (i, 0)),
        ],
        out_specs=pl.BlockSpec((_MLP_BLK, 128), lambda i: (i, 0)),
        out_shape=jax.ShapeDtypeStruct((ROWS, 128), jnp.float32),
    )(p, hs2, hd2, w2)


def _upd_body(sb_ref, h_ref, a_ref, d_ref, o_ref):
    av = a_ref[...]
    x = h_ref[...] + (av[:_NROWS] + av[_NROWS:]) / d_ref[...]
    mu = x
    dl = x - mu
    var = dl * dl
    y = dl / jnp.sqrt(var + 1e-6) * sb_ref[0, 0] + sb_ref[0, 1]
    o_ref[...] = jnp.maximum(y, 0.0) + jnp.log1p(jnp.exp(-jnp.abs(y)))


_NROWS = N_NODES // 128


def _tc_update(lslb, h2, agg, d2):
    return pl.pallas_call(
        _upd_body,
        in_specs=[
            pl.BlockSpec(memory_space=pltpu.SMEM),
            pl.BlockSpec((_NROWS, 128), lambda: (0, 0)),
            pl.BlockSpec((2 * _NROWS, 128), lambda: (0, 0)),
            pl.BlockSpec((_NROWS, 128), lambda: (0, 0)),
        ],
        out_specs=pl.BlockSpec((_NROWS, 128), lambda: (0, 0)),
        out_shape=jax.ShapeDtypeStruct((_NROWS, 128), jnp.float32),
    )(lslb, h2, agg, d2)


def kernel(eps_2d, esrc, edst, ew, ndeg,
           W1_0, b1_0, W2_0, b2_0, ln_s_0, ln_b_0,
           W1_1, b1_1, W2_1, b2_1, ln_s_1, ln_b_1):
    h0 = eps_2d.reshape(-1)
    ew2 = ew.reshape(ROWS, 128)
    edst3 = edst.reshape(NW, ROWS_PER_W, 128)
    edst4 = edst.reshape(NS * NCHUNKS, SCH, 128)
    d2 = ndeg.reshape(_NROWS, 128)

    p0 = jnp.stack([W1_0[0], W1_0[1], W1_0[2], b1_0, 0.5 * W2_0[:, 0],
                    jnp.broadcast_to(b2_0, (HIDDEN,))])
    p1 = jnp.stack([W1_1[0], W1_1[1], W1_1[2], b1_1, 0.5 * W2_1[:, 0],
                    jnp.broadcast_to(b2_1, (HIDDEN,))])
    lslb0 = jnp.concatenate([jnp.broadcast_to(ln_s_0, (LANES,)),
                             jnp.broadcast_to(ln_b_0, (LANES,))])
    lslb1 = jnp.stack([ln_s_1[0], ln_b_1[0]]).reshape(1, 2)

    hs1, hd1 = _sc_gather(h0, esrc, edst)
    m1 = _tc_mlp(p0, hs1.reshape(ROWS, 128), hd1.reshape(ROWS, 128), ew2)
    hs2, hd2, h1r = _sc_mid(m1.reshape(NS * NCHUNKS, SCH, 128), edst4,
                            h0, ndeg, esrc, edst, lslb0)
    m2 = _tc_mlp(p1, hs2.reshape(ROWS, 128), hd2.reshape(ROWS, 128), ew2)
    agg = _sc_scatter(m2.reshape(NW, ROWS_PER_W, 128), edst3)
    out = _tc_update(lslb1, h1r[0].reshape(_NROWS, 128),
                     agg.reshape(2 * _NROWS, 128), d2)
    return out.reshape(N_GRID, N_GRID)
